# baseline (speedup 1.0000x reference)
import jax
import jax.numpy as jnp
from jax import lax
from jax.experimental import pallas as pl
from jax.experimental.pallas import tpu as pltpu
from jax.experimental.pallas import tpu_sc as plsc

BATCH = 16384
EMBED_DIM = 32

_NC = 2
_NS = 16
_NW = _NC * _NS
_B_PER_W = BATCH // _NW
_CHUNK = 128
_NCHUNK = _B_PER_W // _CHUNK


def _gather_body(stars_idx_hbm, pc_idx_hbm, w_stars_hbm, w_pc_hbm, out_hbm,
                 idx_s, idx_p, rows_s, rows_p, sem):
    wid = lax.axis_index("s") * _NC + lax.axis_index("c")
    base = wid * _B_PER_W

    pltpu.sync_copy(stars_idx_hbm.at[wid], idx_s)
    pltpu.sync_copy(pc_idx_hbm.at[wid], idx_p)

    copies = []
    for j in range(_NCHUNK):
        rsl = pl.ds(j * _CHUNK, _CHUNK)
        copies.append(pltpu.async_copy(
            w_stars_hbm.at[idx_s.at[j]], rows_s.at[rsl], sem))
        copies.append(pltpu.async_copy(
            w_pc_hbm.at[idx_p.at[j]], rows_p.at[rsl], sem))
    for c in copies:
        c.wait()

    pltpu.sync_copy(rows_s, out_hbm.at[pl.ds(base, _B_PER_W),
                                       pl.ds(0, EMBED_DIM)])
    pltpu.sync_copy(rows_p, out_hbm.at[pl.ds(base, _B_PER_W),
                                       pl.ds(EMBED_DIM, EMBED_DIM)])


@jax.jit
def _run(stars_idx, postalcode_idx, W_stars, W_postalcode):
    mesh = plsc.VectorSubcoreMesh(core_axis_name="c", subcore_axis_name="s")
    k = pl.kernel(
        _gather_body,
        out_type=jax.ShapeDtypeStruct((BATCH, 2 * EMBED_DIM), jnp.float32),
        mesh=mesh,
        scratch_types=[
            pltpu.VMEM((_NCHUNK, _CHUNK), jnp.int32),
            pltpu.VMEM((_NCHUNK, _CHUNK), jnp.int32),
            pltpu.VMEM((_B_PER_W, EMBED_DIM), jnp.float32),
            pltpu.VMEM((_B_PER_W, EMBED_DIM), jnp.float32),
            pltpu.SemaphoreType.DMA,
        ],
        compiler_params=pltpu.CompilerParams(use_tc_tiling_on_sc=False),
    )
    s_idx = stars_idx.astype(jnp.int32).reshape(_NW, _NCHUNK, _CHUNK)
    p_idx = postalcode_idx.astype(jnp.int32).reshape(_NW, _NCHUNK, _CHUNK)
    return k(s_idx, p_idx, W_stars, W_postalcode)


def kernel(stars_idx, postalcode_idx, W_stars, W_postalcode):
    return _run(stars_idx, postalcode_idx, W_stars, W_postalcode)
